# baseline (speedup 1.0000x reference)
import jax
import jax.numpy as jnp
from jax.experimental import pallas as pl
from jax.experimental.pallas import tpu as pltpu

N = 5000
D = 12544
H = 1024
BM = 200


def _body(xa_ref, xb_ref, w1_ref, b1_ref, w2_ref, b2_ref, w3_ref, b3_ref,
          w4_ref, b4_ref, cls_ref, box_ref):
    xblk = jnp.concatenate(
        [xa_ref[...].astype(jnp.bfloat16),
         xb_ref[...].reshape(BM, D // 2).astype(jnp.bfloat16)], axis=1)
    h1 = jnp.maximum(
        jnp.dot(xblk, w1_ref[...],
                preferred_element_type=jnp.float32) + b1_ref[...], 0.0
    ).astype(jnp.bfloat16)
    h2 = jnp.maximum(
        jnp.dot(h1, w2_ref[...], preferred_element_type=jnp.float32)
        + b2_ref[...], 0.0).astype(jnp.bfloat16)
    logits = jnp.dot(h2, w3_ref[...],
                     preferred_element_type=jnp.float32) + b3_ref[...]
    m = jnp.max(logits, axis=-1, keepdims=True)
    e = jnp.exp(logits - m)
    cls_ref[...] = e / jnp.sum(e, axis=-1, keepdims=True)
    box_ref[...] = jnp.dot(h2, w4_ref[...],
                           preferred_element_type=jnp.float32) + b4_ref[...]


def kernel(feature_vectors, W1, b1, W2, b2, W3, b3, W4, b4):
    C1 = W3.shape[1]
    C4 = W4.shape[1]
    out = pl.pallas_call(
        _body,
        grid=(N // BM,),
        in_specs=[
            pl.BlockSpec((BM, D // 2), lambda i: (i, 0)),
            pl.BlockSpec((BM, 1, 2, D // 4), lambda i: (i, 1, 0, 0)),
            pl.BlockSpec((D, H), lambda i: (0, 0)),
            pl.BlockSpec((1, H), lambda i: (0, 0)),
            pl.BlockSpec((H, H), lambda i: (0, 0)),
            pl.BlockSpec((1, H), lambda i: (0, 0)),
            pl.BlockSpec((H, C1), lambda i: (0, 0)),
            pl.BlockSpec((1, C1), lambda i: (0, 0)),
            pl.BlockSpec((H, C4), lambda i: (0, 0)),
            pl.BlockSpec((1, C4), lambda i: (0, 0)),
        ],
        out_specs=[
            pl.BlockSpec((BM, C1), lambda i: (i, 0)),
            pl.BlockSpec((BM, C4), lambda i: (i, 0)),
        ],
        out_shape=[
            jax.ShapeDtypeStruct((N, C1), jnp.float32),
            jax.ShapeDtypeStruct((N, C4), jnp.float32),
        ],
        compiler_params=pltpu.CompilerParams(
            dimension_semantics=("arbitrary",),
            vmem_limit_bytes=64 * 1024 * 1024,
        ),
    )(feature_vectors, feature_vectors.reshape(N, 2, 2, D // 4),
      W1.astype(jnp.bfloat16), b1.reshape(1, H),
      W2.astype(jnp.bfloat16), b2.reshape(1, H),
      W3.astype(jnp.bfloat16), b3.reshape(1, C1),
      W4.astype(jnp.bfloat16), b4.reshape(1, C4))
    return (out[0], out[1])


# speedup vs baseline: 4.2551x; 4.2551x over previous
import jax
import jax.numpy as jnp
from jax import lax
from jax.experimental import pallas as pl
from jax.experimental.pallas import tpu as pltpu

N = 5000
D = 12544
H = 1024
BM = 200
MBLKS = N // BM
NCH = 5
CH = BM // NCH


def _body(x_hbm, w1_ref, b1_ref, w2_ref, b2_ref, w3_ref, b3_ref,
          w4_ref, b4_ref, cls_ref, box_ref, xbuf, sems):
    i = pl.program_id(0)

    def fire(blk, slot):
        base = blk * BM
        for c in range(NCH):
            pltpu.make_async_copy(
                x_hbm.at[pl.ds(base + c * CH, CH), :],
                xbuf.at[pl.ds(slot * BM + c * CH, CH), :],
                sems.at[slot, c]).start()

    def drain(blk, slot):
        base = blk * BM
        for c in range(NCH):
            pltpu.make_async_copy(
                x_hbm.at[pl.ds(base + c * CH, CH), :],
                xbuf.at[pl.ds(slot * BM + c * CH, CH), :],
                sems.at[slot, c]).wait()

    even = lax.rem(i, 2) == 0

    @pl.when(i == 0)
    def _():
        fire(i, 0)

    @pl.when((i + 1 < MBLKS) & even)
    def _():
        fire(i + 1, 1)

    @pl.when((i + 1 < MBLKS) & jnp.logical_not(even))
    def _():
        fire(i + 1, 0)

    @pl.when(even)
    def _():
        drain(i, 0)

    @pl.when(jnp.logical_not(even))
    def _():
        drain(i, 1)

    slot = lax.rem(i, 2)
    xblk = xbuf[pl.ds(slot * BM, BM), :].astype(jnp.bfloat16)
    h1 = jnp.maximum(
        jnp.dot(xblk, w1_ref[...],
                preferred_element_type=jnp.float32) + b1_ref[...], 0.0
    ).astype(jnp.bfloat16)
    h2 = jnp.maximum(
        jnp.dot(h1, w2_ref[...], preferred_element_type=jnp.float32)
        + b2_ref[...], 0.0).astype(jnp.bfloat16)
    logits = jnp.dot(h2, w3_ref[...],
                     preferred_element_type=jnp.float32) + b3_ref[...]
    m = jnp.max(logits, axis=-1, keepdims=True)
    e = jnp.exp(logits - m)
    cls_ref[...] = e / jnp.sum(e, axis=-1, keepdims=True)
    box_ref[...] = jnp.dot(h2, w4_ref[...],
                           preferred_element_type=jnp.float32) + b4_ref[...]


def kernel(feature_vectors, W1, b1, W2, b2, W3, b3, W4, b4):
    C1 = W3.shape[1]
    C4 = W4.shape[1]
    out = pl.pallas_call(
        _body,
        grid=(MBLKS,),
        in_specs=[
            pl.BlockSpec(memory_space=pltpu.MemorySpace.HBM),
            pl.BlockSpec((D, H), lambda i: (0, 0)),
            pl.BlockSpec((1, H), lambda i: (0, 0)),
            pl.BlockSpec((H, H), lambda i: (0, 0)),
            pl.BlockSpec((1, H), lambda i: (0, 0)),
            pl.BlockSpec((H, C1), lambda i: (0, 0)),
            pl.BlockSpec((1, C1), lambda i: (0, 0)),
            pl.BlockSpec((H, C4), lambda i: (0, 0)),
            pl.BlockSpec((1, C4), lambda i: (0, 0)),
        ],
        out_specs=[
            pl.BlockSpec((BM, C1), lambda i: (i, 0)),
            pl.BlockSpec((BM, C4), lambda i: (i, 0)),
        ],
        out_shape=[
            jax.ShapeDtypeStruct((N, C1), jnp.float32),
            jax.ShapeDtypeStruct((N, C4), jnp.float32),
        ],
        scratch_shapes=[
            pltpu.VMEM((2 * BM, D), jnp.float32),
            pltpu.SemaphoreType.DMA((2, NCH)),
        ],
        compiler_params=pltpu.CompilerParams(
            dimension_semantics=("arbitrary",),
            vmem_limit_bytes=64 * 1024 * 1024,
        ),
    )(feature_vectors, W1.astype(jnp.bfloat16), b1.reshape(1, H),
      W2.astype(jnp.bfloat16), b2.reshape(1, H),
      W3.astype(jnp.bfloat16), b3.reshape(1, C1),
      W4.astype(jnp.bfloat16), b4.reshape(1, C4))
    return (out[0], out[1])


# speedup vs baseline: 4.5193x; 1.0621x over previous
import jax
import jax.numpy as jnp
from jax import lax
from jax.experimental import pallas as pl
from jax.experimental.pallas import tpu as pltpu

N = 5000
D = 12544
H = 1024
BM = 1000
BK = 1792
KBLKS = D // BK
MBLKS = N // BM
STEPS = KBLKS * MBLKS
NCH = 5
CH = BM // NCH


def _body(x_hbm, w1_ref, b1_ref, w2_ref, b2_ref, w3_ref, b3_ref,
          w4_ref, b4_ref, cls_ref, box_ref, xbuf, acc_ref, sems):
    k = pl.program_id(0)
    i = pl.program_id(1)
    s = k * MBLKS + i

    def copies(kk, ii, slot):
        for c in range(NCH):
            yield pltpu.make_async_copy(
                x_hbm.at[pl.ds(ii * BM + c * CH, CH), pl.ds(kk * BK, BK)],
                xbuf.at[pl.ds(slot * BM + c * CH, CH), :],
                sems.at[slot, c])

    def fire(kk, ii, slot):
        for cp in copies(kk, ii, slot):
            cp.start()

    def drain(kk, ii, slot):
        for cp in copies(kk, ii, slot):
            cp.wait()

    s1 = s + 1
    k1 = s1 // MBLKS
    i1 = lax.rem(s1, MBLKS)
    even = lax.rem(s, 2) == 0

    @pl.when(s == 0)
    def _():
        fire(k, i, 0)

    @pl.when((s1 < STEPS) & even)
    def _():
        fire(k1, i1, 1)

    @pl.when((s1 < STEPS) & jnp.logical_not(even))
    def _():
        fire(k1, i1, 0)

    @pl.when(even)
    def _():
        drain(k, i, 0)

    @pl.when(jnp.logical_not(even))
    def _():
        drain(k, i, 1)

    slot = lax.rem(s, 2)
    rows = pl.ds(i * BM, BM)
    xblk = xbuf[pl.ds(slot * BM, BM), :].astype(jnp.bfloat16)
    part = jnp.dot(xblk, w1_ref[...].astype(jnp.bfloat16),
                   preferred_element_type=jnp.float32)

    @pl.when(k == 0)
    def _():
        acc_ref[rows, :] = part

    @pl.when(k > 0)
    def _():
        acc_ref[rows, :] += part

    @pl.when(k == KBLKS - 1)
    def _():
        h1 = jnp.maximum(acc_ref[rows, :] + b1_ref[...], 0.0
                         ).astype(jnp.bfloat16)
        h2 = jnp.maximum(
            jnp.dot(h1, w2_ref[...], preferred_element_type=jnp.float32)
            + b2_ref[...], 0.0).astype(jnp.bfloat16)
        logits = jnp.dot(h2, w3_ref[...],
                         preferred_element_type=jnp.float32) + b3_ref[...]
        m = jnp.max(logits, axis=-1, keepdims=True)
        e = jnp.exp(logits - m)
        cls_ref[...] = e / jnp.sum(e, axis=-1, keepdims=True)
        box_ref[...] = jnp.dot(h2, w4_ref[...],
                               preferred_element_type=jnp.float32) + b4_ref[...]


def kernel(feature_vectors, W1, b1, W2, b2, W3, b3, W4, b4):
    C1 = W3.shape[1]
    C4 = W4.shape[1]
    grid = (KBLKS, MBLKS)
    out = pl.pallas_call(
        _body,
        grid=grid,
        in_specs=[
            pl.BlockSpec(memory_space=pltpu.MemorySpace.HBM),
            pl.BlockSpec((BK, H), lambda k, i: (k, 0)),
            pl.BlockSpec((1, H), lambda k, i: (0, 0)),
            pl.BlockSpec((H, H), lambda k, i: (0, 0)),
            pl.BlockSpec((1, H), lambda k, i: (0, 0)),
            pl.BlockSpec((H, C1), lambda k, i: (0, 0)),
            pl.BlockSpec((1, C1), lambda k, i: (0, 0)),
            pl.BlockSpec((H, C4), lambda k, i: (0, 0)),
            pl.BlockSpec((1, C4), lambda k, i: (0, 0)),
        ],
        out_specs=[
            pl.BlockSpec((BM, C1), lambda k, i: (i, 0)),
            pl.BlockSpec((BM, C4), lambda k, i: (i, 0)),
        ],
        out_shape=[
            jax.ShapeDtypeStruct((N, C1), jnp.float32),
            jax.ShapeDtypeStruct((N, C4), jnp.float32),
        ],
        scratch_shapes=[
            pltpu.VMEM((2 * BM, BK), jnp.float32),
            pltpu.VMEM((N, H), jnp.float32),
            pltpu.SemaphoreType.DMA((2, NCH)),
        ],
        compiler_params=pltpu.CompilerParams(
            dimension_semantics=("arbitrary", "arbitrary"),
            vmem_limit_bytes=64 * 1024 * 1024,
        ),
    )(feature_vectors, W1, b1.reshape(1, H),
      W2.astype(jnp.bfloat16), b2.reshape(1, H),
      W3.astype(jnp.bfloat16), b3.reshape(1, C1),
      W4.astype(jnp.bfloat16), b4.reshape(1, C4))
    return (out[0], out[1])
